# baseline (speedup 1.0000x reference)
import functools

import jax
import jax.numpy as jnp
from jax.experimental import pallas as pl
from jax.experimental.pallas import tpu as pltpu

IB = 128


def _body(NI, S, Dm, q_ref, k_ref, v_ref, wq_ref, wk_ref, wv_ref, wo_ref,
          bqk_ref, bv_ref, bo_ref, w2_ref, out_ref, attn_ref,
          qts3, kts, vps, sc):
    f32 = jnp.float32
    bf16 = jnp.bfloat16
    c = pl.program_id(1)

    @pl.when(c == 0)
    def _proj():
        for n in range(NI):
            qts3[n] = jax.lax.dot_general(
                q_ref[0, n * IB:(n + 1) * IB, :], wq_ref[...],
                (((1,), (1,)), ((), ())),
                preferred_element_type=f32).astype(bf16)
        kts[...] = (jax.lax.dot_general(
            k_ref[0], wk_ref[...], (((1,), (1,)), ((), ())),
            preferred_element_type=f32) + bqk_ref[...]).astype(bf16)
        vps[...] = jax.lax.dot_general(
            v_ref[0], wv_ref[...], (((1,), (1,)), ((), ())),
            preferred_element_type=f32) + bv_ref[...]

    wsb = w2_ref[...].astype(bf16)
    for g in range(IB // 2):
        t2 = jnp.concatenate(
            [jnp.tanh(kts[...] + qts3[c, 2 * g + m:2 * g + m + 1, :])
             for m in range(2)], axis=1)
        sc[2 * g:2 * g + 2, :] = jax.lax.dot_general(
            wsb, t2, (((1,), (1,)), ((), ())), preferred_element_type=f32)

    s = sc[...]
    m = jnp.max(s, axis=1, keepdims=True)
    e = jnp.exp(s - m)
    den = jnp.sum(e, axis=1, keepdims=True)
    p = e / den
    attn_ref[0, 0] = p
    av = jax.lax.dot_general(p, vps[...], (((1,), (0,)), ((), ())),
                             preferred_element_type=f32)
    out_ref[0] = jax.lax.dot_general(
        av, wo_ref[...], (((1,), (1,)), ((), ())),
        preferred_element_type=f32) + bo_ref[...]


def _fwd(query, key, value, Wq, bq, Wk, bk, Wv, bv, Ws, bs, Wo, bo,
         interpret=False):
    f32 = jnp.float32
    B, S, Dm = query.shape
    NI = S // IB
    body = functools.partial(_body, NI, S, Dm)

    in_specs = [
        pl.BlockSpec((1, S, Dm), lambda b, c: (b, 0, 0)),
        pl.BlockSpec((1, S, Dm), lambda b, c: (b, 0, 0)),
        pl.BlockSpec((1, S, Dm), lambda b, c: (b, 0, 0)),
        pl.BlockSpec((Dm, Dm), lambda b, c: (0, 0)),
        pl.BlockSpec((Dm, Dm), lambda b, c: (0, 0)),
        pl.BlockSpec((Dm, Dm), lambda b, c: (0, 0)),
        pl.BlockSpec((Dm, Dm), lambda b, c: (0, 0)),
        pl.BlockSpec((1, Dm), lambda b, c: (0, 0)),
        pl.BlockSpec((1, Dm), lambda b, c: (0, 0)),
        pl.BlockSpec((1, Dm), lambda b, c: (0, 0)),
        pl.BlockSpec((2, 2 * Dm), lambda b, c: (0, 0)),
    ]
    out_specs = [
        pl.BlockSpec((1, IB, Dm), lambda b, c: (b, c, 0)),
        pl.BlockSpec((1, 1, IB, S), lambda b, c: (b, 0, c, 0)),
    ]
    out_shape = [
        jax.ShapeDtypeStruct((B, S, Dm), f32),
        jax.ShapeDtypeStruct((B, 1, S, S), f32),
    ]
    scratch = [
        pltpu.VMEM((NI, IB, Dm), jnp.bfloat16),
        pltpu.VMEM((S, Dm), jnp.bfloat16),
        pltpu.VMEM((S, Dm), f32),
        pltpu.VMEM((IB, S), f32),
    ]
    out, attn = pl.pallas_call(
        body,
        grid=(B, NI),
        in_specs=in_specs,
        out_specs=out_specs,
        out_shape=out_shape,
        scratch_shapes=scratch,
        compiler_params=pltpu.CompilerParams(
            dimension_semantics=("parallel", "arbitrary"),
            vmem_limit_bytes=48 * 1024 * 1024,
        ),
        interpret=interpret,
    )(query, key, value, Wq, Wk, Wv, Wo,
      (bq + bk).reshape(1, Dm), bv.reshape(1, Dm), bo.reshape(1, Dm),
      jnp.kron(jnp.eye(2, dtype=f32), Ws.reshape(1, Dm)))
    return out, attn


def kernel(query, key, value, Wq, bq, Wk, bk, Wv, bv, Ws, bs, Wo, bo):
    return _fwd(query, key, value, Wq, bq, Wk, bk, Wv, bv, Ws, bs, Wo, bo)


# speedup vs baseline: 1.0125x; 1.0125x over previous
import functools

import jax
import jax.numpy as jnp
from jax.experimental import pallas as pl
from jax.experimental.pallas import tpu as pltpu

IB = 128


def _body(NI, S, Dm, q_ref, k_ref, v_ref, wq_ref, wk_ref, wv_ref, wo_ref,
          bqk_ref, bv_ref, bo_ref, w2_ref, out_ref, attn_ref,
          qts3, kts, vps, sc):
    f32 = jnp.float32
    bf16 = jnp.bfloat16
    c = pl.program_id(1)

    @pl.when(c == 0)
    def _proj():
        for n in range(NI):
            qts3[n] = jax.lax.dot_general(
                wq_ref[...], q_ref[0, n * IB:(n + 1) * IB, :],
                (((1,), (1,)), ((), ())),
                preferred_element_type=f32).astype(bf16)
        kts[...] = (jax.lax.dot_general(
            wk_ref[...], k_ref[0], (((1,), (1,)), ((), ())),
            preferred_element_type=f32) + bqk_ref[...]).astype(bf16)
        vps[...] = jax.lax.dot_general(
            v_ref[0], wv_ref[...], (((1,), (1,)), ((), ())),
            preferred_element_type=f32) + bv_ref[...]

    w2b = w2_ref[...].astype(bf16)
    qtb = qts3[c]
    kf = kts[...]
    for g in range(IB // 8):
        slabs = [jnp.tanh(qtb[:, g * 8 + m:g * 8 + m + 1] + kf)
                 for m in range(8)]
        t8 = jnp.concatenate(slabs, axis=0)
        sc[g * 8:(g + 1) * 8, :] = jax.lax.dot_general(
            w2b, t8, (((1,), (0,)), ((), ())), preferred_element_type=f32)

    s = sc[...]
    m = jnp.max(s, axis=1, keepdims=True)
    e = jnp.exp(s - m)
    den = jnp.sum(e, axis=1, keepdims=True)
    p = e / den
    attn_ref[0, 0] = p
    av = jax.lax.dot_general(p, vps[...], (((1,), (0,)), ((), ())),
                             preferred_element_type=f32)
    out_ref[0] = jax.lax.dot_general(
        av, wo_ref[...], (((1,), (1,)), ((), ())),
        preferred_element_type=f32) + bo_ref[...]


def _fwd(query, key, value, Wq, bq, Wk, bk, Wv, bv, Ws, bs, Wo, bo,
         interpret=False):
    f32 = jnp.float32
    B, S, Dm = query.shape
    NI = S // IB
    body = functools.partial(_body, NI, S, Dm)

    in_specs = [
        pl.BlockSpec((1, S, Dm), lambda b, c: (b, 0, 0)),
        pl.BlockSpec((1, S, Dm), lambda b, c: (b, 0, 0)),
        pl.BlockSpec((1, S, Dm), lambda b, c: (b, 0, 0)),
        pl.BlockSpec((Dm, Dm), lambda b, c: (0, 0)),
        pl.BlockSpec((Dm, Dm), lambda b, c: (0, 0)),
        pl.BlockSpec((Dm, Dm), lambda b, c: (0, 0)),
        pl.BlockSpec((Dm, Dm), lambda b, c: (0, 0)),
        pl.BlockSpec((Dm, 1), lambda b, c: (0, 0)),
        pl.BlockSpec((1, Dm), lambda b, c: (0, 0)),
        pl.BlockSpec((1, Dm), lambda b, c: (0, 0)),
        pl.BlockSpec((8, 8 * Dm), lambda b, c: (0, 0)),
    ]
    out_specs = [
        pl.BlockSpec((1, IB, Dm), lambda b, c: (b, c, 0)),
        pl.BlockSpec((1, 1, IB, S), lambda b, c: (b, 0, c, 0)),
    ]
    out_shape = [
        jax.ShapeDtypeStruct((B, S, Dm), f32),
        jax.ShapeDtypeStruct((B, 1, S, S), f32),
    ]
    scratch = [
        pltpu.VMEM((NI, Dm, IB), jnp.bfloat16),
        pltpu.VMEM((Dm, S), jnp.bfloat16),
        pltpu.VMEM((S, Dm), f32),
        pltpu.VMEM((IB, S), f32),
    ]
    out, attn = pl.pallas_call(
        body,
        grid=(B, NI),
        in_specs=in_specs,
        out_specs=out_specs,
        out_shape=out_shape,
        scratch_shapes=scratch,
        compiler_params=pltpu.CompilerParams(
            dimension_semantics=("parallel", "arbitrary"),
            vmem_limit_bytes=48 * 1024 * 1024,
        ),
        interpret=interpret,
    )(query, key, value, Wq, Wk, Wv, Wo,
      (bq + bk).reshape(Dm, 1), bv.reshape(1, Dm), bo.reshape(1, Dm),
      jnp.kron(jnp.eye(8, dtype=f32), Ws.reshape(1, Dm)))
    return out, attn


def kernel(query, key, value, Wq, bq, Wk, bk, Wv, bv, Ws, bs, Wo, bo):
    return _fwd(query, key, value, Wq, bq, Wk, bk, Wv, bv, Ws, bs, Wo, bo)
